# Initial kernel scaffold; baseline (speedup 1.0000x reference)
import jax
import jax.numpy as jnp
from jax.experimental import pallas as pl


def kernel(x, lengths, ln1_g, ln1_b, W_emb, b_emb, ln2_g, ln2_b, pos, ln_a_g, ln_a_b, A, Bm, Cf, Cb, Dm, ln_f_g, ln_f_b, W1, b1, W2, b2):
    raise NotImplementedError("write your pallas kernel here")


# speedup vs baseline: 29.6314x; 29.6314x over previous
import jax
import jax.numpy as jnp
from jax.experimental import pallas as pl
from jax.experimental.pallas import tpu as pltpu

_B, _T, _D, _H, _N, _L = 16, 2048, 128, 512, 64, 6


def _lnorm(v, g, b):
    m = jnp.mean(v, axis=-1, keepdims=True)
    c = v - m
    s = jnp.mean(c * c, axis=-1, keepdims=True)
    return c * jax.lax.rsqrt(s + 1e-5) * g + b


def _biscan(u, a):
    t = u.shape[0]
    yf = u
    yb = u
    ap = a
    s = 1
    while s < t:
        zpad = jnp.zeros((s, u.shape[1]), u.dtype)
        yf = yf + ap * jnp.concatenate([zpad, yf[: t - s, :]], axis=0)
        yb = yb + ap * jnp.concatenate([yb[s:, :], zpad], axis=0)
        ap = ap * ap
        s *= 2
    return yf, yb


def _kern(len_ref, x_ref, pos_ref, W_emb_ref, b_emb_ref, ln1_g_ref, ln1_b_ref,
          ln2_g_ref, ln2_b_ref, ln_a_g_ref, ln_a_b_ref, A_ref, Bm_ref,
          Cfb_ref, Dm_ref, ln_f_g_ref, ln_f_b_ref, W1_ref, b1_ref, W2_ref,
          b2_ref, out_ref):
    b = pl.program_id(0)
    length = len_ref[b]
    row = jax.lax.broadcasted_iota(jnp.int32, (_T, 1), 0)
    mask = (row < length).astype(jnp.float32)

    @pl.when(pl.program_id(1) == 0)
    def _embed():
        xv = x_ref[0] * mask
        h0 = jnp.dot(_lnorm(xv, ln1_g_ref[0:1, :], ln1_b_ref[0:1, :]),
                     W_emb_ref[...], preferred_element_type=jnp.float32)
        h0 = h0 + b_emb_ref[0:1, :]
        h0 = _lnorm(h0, ln2_g_ref[0:1, :], ln2_b_ref[0:1, :])
        out_ref[0] = (h0 + pos_ref[0]) * mask

    h = out_ref[0]
    z = _lnorm(h, ln_a_g_ref[0], ln_a_b_ref[0])
    u = jnp.dot(z, Bm_ref[0], preferred_element_type=jnp.float32)
    hf, hb = _biscan(u, A_ref[0])
    hfb = jnp.concatenate([hf, hb], axis=1)
    y = jnp.dot(hfb, Cfb_ref[0], preferred_element_type=jnp.float32)
    y = y + z * Dm_ref[0]
    h = h + jax.nn.gelu(y) * mask
    z2 = _lnorm(h, ln_f_g_ref[0], ln_f_b_ref[0])
    f = jnp.dot(z2, W1_ref[0], preferred_element_type=jnp.float32)
    f = jax.nn.gelu(f + b1_ref[0])
    f = jnp.dot(f, W2_ref[0], preferred_element_type=jnp.float32)
    f = f + b2_ref[0]
    out_ref[0] = h + f * mask


def kernel(x, lengths, ln1_g, ln1_b, W_emb, b_emb, ln2_g, ln2_b, pos, ln_a_g,
           ln_a_b, A, Bm, Cf, Cb, Dm, ln_f_g, ln_f_b, W1, b1, W2, b2):
    Cfb = jnp.concatenate([Cf, Cb], axis=1)
    full = lambda shape: pl.BlockSpec(shape, lambda b, l: (0,) * len(shape))
    perl2 = lambda d: pl.BlockSpec((1, 1, d), lambda b, l: (l, 0, 0))
    perl3 = lambda d0, d1: pl.BlockSpec((1, d0, d1), lambda b, l: (l, 0, 0))

    out = pl.pallas_call(
        _kern,
        grid=(_B, _L),
        in_specs=[
            pl.BlockSpec(memory_space=pltpu.SMEM),
            pl.BlockSpec((1, _T, _D), lambda b, l: (b, 0, 0)),
            pl.BlockSpec((1, _T, _H), lambda b, l: (0, 0, 0)),
            full((_D, _H)),
            full((1, _H)),
            full((1, _D)), full((1, _D)),
            full((1, _H)), full((1, _H)),
            perl2(_H), perl2(_H),
            perl2(_N),
            perl3(_H, _N),
            perl3(2 * _N, _H),
            perl2(_H),
            perl2(_H), perl2(_H),
            perl3(_H, 2 * _H),
            perl2(2 * _H),
            perl3(2 * _H, _H),
            perl2(_H),
        ],
        out_specs=pl.BlockSpec((1, _T, _H), lambda b, l: (b, 0, 0)),
        out_shape=jax.ShapeDtypeStruct((_B, _T, _H), jnp.float32),
        compiler_params=pltpu.CompilerParams(
            dimension_semantics=("parallel", "arbitrary"),
            vmem_limit_bytes=110 * 1024 * 1024,
        ),
    )(lengths, x, pos, W_emb, b_emb.reshape(1, _H),
      ln1_g.reshape(1, _D), ln1_b.reshape(1, _D),
      ln2_g.reshape(1, _H), ln2_b.reshape(1, _H),
      ln_a_g.reshape(_L, 1, _H), ln_a_b.reshape(_L, 1, _H),
      A.reshape(_L, 1, _N), Bm, Cfb, Dm.reshape(_L, 1, _H),
      ln_f_g.reshape(_L, 1, _H), ln_f_b.reshape(_L, 1, _H),
      W1, b1.reshape(_L, 1, 2 * _H), W2, b2.reshape(_L, 1, _H))
    return out
